# baseline (speedup 1.0000x reference)
import functools

import jax
import jax.numpy as jnp
from jax import lax
from jax.experimental import pallas as pl
from jax.experimental.pallas import tpu as pltpu
from jax.experimental.pallas import tpu_sc as plsc

R = 1024
V = 100000
NC = 2
NS = 16
NW = NC * NS
RPW = R // NW
C = 20000
NCHUNK = V // C
VPC = C // 16
UNROLL = 10
NT = RPW * NCHUNK

_mesh = plsc.VectorSubcoreMesh(
    core_axis_name="c", subcore_axis_name="s", num_cores=NC, num_subcores=NS)


@functools.partial(
    pl.kernel,
    out_type=jax.ShapeDtypeStruct((R,), jnp.int32),
    mesh=_mesh,
    scratch_types=[
        pltpu.VMEM((2, C), jnp.float32),
        pltpu.VMEM((RPW,), jnp.int32),
        pltpu.SemaphoreType.DMA,
        pltpu.SemaphoreType.DMA,
    ],
    compiler_params=pltpu.CompilerParams(
        use_tc_tiling_on_sc=False, needs_layout_passes=False),
)
def _argmax_sc(x_hbm, out_hbm, buf, res, sem0, sem1):
    wid = lax.axis_index("s") * NC + lax.axis_index("c")
    row0 = wid * RPW
    sems = (sem0, sem1)

    neg_inf = jnp.full((16,), -jnp.inf, dtype=jnp.float32)
    zeros = jnp.zeros((16,), dtype=jnp.int32)
    lane = lax.broadcasted_iota(jnp.int32, (16,), 0)

    pltpu.async_copy(x_hbm.at[row0, pl.ds(0, C)], buf.at[0], sem0)
    pltpu.async_copy(x_hbm.at[row0, pl.ds(C, C)], buf.at[1], sem1)

    def pair_body(_, carry):
        maxv, idxv, row, c, prow, pc = carry
        for b in range(2):
            first = c == 0
            maxv = jnp.where(first, neg_inf, maxv)
            idxv = jnp.where(first, zeros, idxv)

            pltpu.make_async_copy(
                x_hbm.at[row0, pl.ds(0, C)], buf.at[b], sems[b]).wait()

            base = c * VPC

            def vec_body(iu, vc):
                mv, iv = vc
                i0 = iu * UNROLL
                for u in range(UNROLL):
                    i = i0 + u
                    v = buf[b, pl.ds(i * 16, 16)]
                    vid = jnp.full((16,), base + i, dtype=jnp.int32)
                    upd = v > mv
                    mv = jnp.where(upd, v, mv)
                    iv = jnp.where(upd, vid, iv)
                return (mv, iv)

            maxv, idxv = lax.fori_loop(
                0, VPC // UNROLL, vec_body, (maxv, idxv))

            @pl.when(prow < RPW)
            def _():
                pltpu.async_copy(
                    x_hbm.at[row0 + prow, pl.ds(pc * C, C)],
                    buf.at[b], sems[b])

            @pl.when(c == NCHUNK - 1)
            def _():
                m = jnp.max(maxv)
                full = idxv * 16 + lane
                cand = jnp.where(maxv == m, full, jnp.int32(1 << 30))
                best = jnp.min(cand)
                off = row & jnp.int32(~15)
                lanepos = row & jnp.int32(15)
                seg = res[pl.ds(off, 16)]
                seg = jnp.where(lane == lanepos,
                                jnp.full((16,), best, dtype=jnp.int32), seg)
                res[pl.ds(off, 16)] = seg

            one = jnp.int32(1)
            zero = jnp.int32(0)
            c1 = c + one
            wrap = c1 == NCHUNK
            c = jnp.where(wrap, zero, c1)
            row = row + jnp.where(wrap, one, zero)
            pc1 = pc + one
            pwrap = pc1 == NCHUNK
            pc = jnp.where(pwrap, zero, pc1)
            prow = prow + jnp.where(pwrap, one, zero)
        return (maxv, idxv, row, c, prow, pc)

    lax.fori_loop(
        0, NT // 2, pair_body,
        (neg_inf, zeros, jnp.int32(0), jnp.int32(0), jnp.int32(0),
         jnp.int32(2)))

    pltpu.sync_copy(res, out_hbm.at[pl.ds(row0, RPW)])


def kernel(inputs):
    return _argmax_sc(inputs)


# speedup vs baseline: 1.8825x; 1.8825x over previous
import functools

import jax
import jax.numpy as jnp
from jax import lax
from jax.experimental import pallas as pl
from jax.experimental.pallas import tpu as pltpu
from jax.experimental.pallas import tpu_sc as plsc

R = 1024
V = 100000
NC = 2
NS = 16
NW = NC * NS
RPW = R // NW
NG = RPW // 8
VMAIN = 99968
VTAIL = V - VMAIN
CW = 1408
NCHUNK = VMAIN // CW
VPR = CW // 16
NT = NG * NCHUNK
BIG = 1 << 30

_mesh = plsc.VectorSubcoreMesh(
    core_axis_name="c", subcore_axis_name="s", num_cores=NC, num_subcores=NS)


@functools.partial(
    pl.kernel,
    out_type=jax.ShapeDtypeStruct((R,), jnp.int32),
    mesh=_mesh,
    scratch_types=[
        pltpu.VMEM((2, 8, CW), jnp.float32),
        pltpu.VMEM((NG, 8, VTAIL), jnp.float32),
        pltpu.VMEM((RPW,), jnp.int32),
        pltpu.SemaphoreType.DMA,
        pltpu.SemaphoreType.DMA,
        pltpu.SemaphoreType.DMA,
    ],
    compiler_params=pltpu.CompilerParams(needs_layout_passes=False),
)
def _argmax_sc(x_hbm, out_hbm, buf, tails, res, sem0, sem1, sem2):
    wid = lax.axis_index("s") * NC + lax.axis_index("c")
    row0 = wid * RPW
    sems = (sem0, sem1)

    neg_inf = jnp.full((16,), -jnp.inf, dtype=jnp.float32)
    zeros = jnp.zeros((16,), dtype=jnp.int32)
    lane = lax.broadcasted_iota(jnp.int32, (16,), 0)

    for g in range(NG):
        pltpu.async_copy(
            x_hbm.at[pl.ds(row0 + g * 8, 8), pl.ds(VMAIN, VTAIL)],
            tails.at[g], sem2)
    for g in range(NG):
        pltpu.make_async_copy(
            x_hbm.at[pl.ds(row0 + g * 8, 8), pl.ds(VMAIN, VTAIL)],
            tails.at[g], sem2).wait()

    pltpu.async_copy(
        x_hbm.at[pl.ds(row0, 8), pl.ds(0, CW)], buf.at[0], sem0)
    pltpu.async_copy(
        x_hbm.at[pl.ds(row0, 8), pl.ds(CW, CW)], buf.at[1], sem1)

    def pair_body(_, carry):
        mvs, ivs, grp, c, pgrp, pc = carry
        for b in range(2):
            first = c == 0
            mvs = tuple(jnp.where(first, neg_inf, mv) for mv in mvs)
            ivs = tuple(jnp.where(first, zeros, iv) for iv in ivs)

            pltpu.make_async_copy(
                x_hbm.at[pl.ds(row0, 8), pl.ds(0, CW)],
                buf.at[b], sems[b]).wait()

            base = c * VPR

            def vec_body(j2, vc):
                mv, iv = vc
                mv = list(mv)
                iv = list(iv)
                for u in range(2):
                    j = j2 * 2 + u
                    vid = base + j
                    for r in range(8):
                        v = buf[b, r, pl.ds(j * 16, 16)]
                        upd = v > mv[r]
                        mv[r] = jnp.where(upd, v, mv[r])
                        iv[r] = jnp.where(
                            upd, jnp.full((16,), vid, dtype=jnp.int32), iv[r])
                return (tuple(mv), tuple(iv))

            mvs, ivs = lax.fori_loop(0, VPR // 2, vec_body, (mvs, ivs))

            @pl.when(pgrp < NG)
            def _():
                pltpu.async_copy(
                    x_hbm.at[pl.ds(row0 + pgrp * 8, 8), pl.ds(pc * CW, CW)],
                    buf.at[b], sems[b])

            @pl.when(c == NCHUNK - 1)
            def _():
                seg_off = jnp.where(grp >= 2, jnp.int32(16), jnp.int32(0))
                seg = res[pl.ds(seg_off, 16)]
                lane8 = jnp.where((grp & jnp.int32(1)) == 1,
                                  jnp.int32(8), jnp.int32(0))
                for r in range(8):
                    mv, iv = mvs[r], ivs[r]
                    for t in range(2):
                        v = tails[grp, r, pl.ds(t * 16, 16)]
                        vid = jnp.full((16,), VMAIN // 16 + t, jnp.int32)
                        upd = v > mv
                        mv = jnp.where(upd, v, mv)
                        iv = jnp.where(upd, vid, iv)
                    m = jnp.max(mv)
                    full = iv * 16 + lane
                    cand = jnp.where(mv == m, full, jnp.int32(BIG))
                    best = jnp.min(cand)
                    seg = jnp.where(lane == lane8 + r,
                                    jnp.full((16,), best, dtype=jnp.int32),
                                    seg)
                res[pl.ds(seg_off, 16)] = seg

            one = jnp.int32(1)
            zero = jnp.int32(0)
            c1 = c + one
            wrap = c1 == NCHUNK
            c = jnp.where(wrap, zero, c1)
            grp = grp + jnp.where(wrap, one, zero)
            pc1 = pc + one
            pwrap = pc1 == NCHUNK
            pc = jnp.where(pwrap, zero, pc1)
            pgrp = pgrp + jnp.where(pwrap, one, zero)
        return (mvs, ivs, grp, c, pgrp, pc)

    init_mvs = tuple(neg_inf for _ in range(8))
    init_ivs = tuple(zeros for _ in range(8))
    lax.fori_loop(
        0, NT // 2, pair_body,
        (init_mvs, init_ivs, jnp.int32(0), jnp.int32(0), jnp.int32(0),
         jnp.int32(2)))

    pltpu.sync_copy(res, out_hbm.at[pl.ds(row0, RPW)])


def kernel(inputs):
    return _argmax_sc(inputs)
